# baseline (speedup 1.0000x reference)
import jax
import jax.numpy as jnp
from jax import lax
from jax.experimental import pallas as pl
from jax.experimental.pallas import tpu as pltpu
from jax.experimental.pallas import tpu_sc as plsc

N = 4096
DF = 128
NNZ_ADJ = 131072
NNZ_D = 16384
NC = 2
NS = 16
NW = NC * NS
L = 16
G = 128
RPT = N // NS

PP = 4
PTD = NNZ_D // NS
PTA = NNZ_ADJ // NW
DGT = PTD // G
AGT = PTA // G

_mesh = plsc.VectorSubcoreMesh(
    core_axis_name="c", subcore_axis_name="s", num_cores=NC, num_subcores=NS
)


def _mega_body(d1r, d1c, d1v, a_r, a_c, a_v, rnd, xb, zeros,
               t_stage, s2_out, ap_out,
               acc, dcol, drow, dval, acol, arow, aval, rndbuf,
               gb0, gb1, bb0, bb1, bb2, bb3,
               gsem0, gsem1, gsem2, gsem3,
               ssem0, ssem1, psem):
    c = lax.axis_index("c")
    s = lax.axis_index("s")
    wid = s * NC + c
    gbufs = (gb0, gb1)
    bbufs = (bb0, bb1, bb2, bb3)
    gsems = (gsem0, gsem1, gsem2, gsem3)
    ssems = (ssem0, ssem1)
    sl = pl.ds(s * RPT, RPT)

    pre = [
        pltpu.async_copy(zeros.at[sl], acc.at[sl], psem),
        pltpu.async_copy(d1c.at[pl.ds(s * DGT, DGT)], dcol, psem),
        pltpu.async_copy(d1r.at[pl.ds(s * DGT, DGT)], drow, psem),
        pltpu.async_copy(d1v.at[pl.ds(pl.multiple_of(s * PTD, 8), PTD)], dval, psem),
        pltpu.async_copy(a_c.at[pl.ds(wid * AGT, AGT)], acol, psem),
        pltpu.async_copy(a_r.at[pl.ds(wid * AGT, AGT)], arow, psem),
        pltpu.async_copy(a_v.at[pl.ds(pl.multiple_of(wid * PTA, 8), PTA)], aval, psem),
        pltpu.async_copy(rnd.at[pl.ds(pl.multiple_of(s * RPT, 8), RPT)], rndbuf, psem),
    ]
    for d in pre:
        d.wait()
    plsc.subcore_barrier()

    def fire_scatter(rowb, idx, p):
        pltpu.async_copy(gbufs[p], acc.at[rowb.at[idx]], ssems[p], add=True)

    def swait(rowb, p):
        pltpu.make_async_copy(gbufs[p], acc.at[rowb.at[0]], ssems[p]).wait()

    def run_spmm_bf(colb, rowb, valb, nch, table):
        assert nch % PP == 0 and nch >= PP

        def fire_gather(idx, p):
            pltpu.async_copy(table.at[colb.at[idx]], bbufs[p], gsems[p])

        def gwait(p):
            pltpu.make_async_copy(table.at[colb.at[0]], bbufs[p], gsems[p]).wait()

        def scale(cur, bp, fp):
            def body(k16, _):
                vv = valb[pl.ds(cur * G + k16 * L, L)]
                for j in range(L):
                    v = vv[j]
                    row = k16 * L + j
                    for q in range(DF // (2 * L)):
                        vi = bbufs[bp][row, pl.ds(q * L, L)]
                        ve = lax.bitcast_convert_type(vi << 16, jnp.float32) * v
                        vo = lax.bitcast_convert_type(vi & (-65536), jnp.float32) * v
                        gbufs[fp][row, pl.ds(q * 2 * L, L)] = ve
                        gbufs[fp][row, pl.ds(q * 2 * L + L, L)] = vo
                return 0

            lax.fori_loop(0, G // L, body, 0)

        fire_gather(0, 0)
        fire_gather(1, 1)

        def group(g, _):
            ci = g * PP
            for pos in range(PP):
                cur = ci + pos
                fp = pos % 2

                @pl.when(cur + 2 < nch)
                def _():
                    fire_gather(cur + 2, (pos + 2) % PP)

                gwait(pos)

                @pl.when(cur >= 2)
                def _():
                    swait(rowb, fp)

                scale(cur, pos, fp)
                fire_scatter(rowb, cur, fp)
            return 0

        lax.fori_loop(0, nch // PP, group, 0)
        swait(rowb, 0)
        swait(rowb, 1)

    def run_spmm_f32(colb, rowb, valb, nch, table):
        assert nch % PP == 0 and nch >= PP

        def fire_gather(idx, p):
            pltpu.async_copy(table.at[colb.at[idx]], gbufs[p], gsems[p])

        def gwait(p):
            pltpu.make_async_copy(table.at[colb.at[0]], gbufs[p], gsems[p]).wait()

        def scale(cur, p):
            def body(k16, _):
                vv = valb[pl.ds(cur * G + k16 * L, L)]
                for j in range(L):
                    v = vv[j]
                    row = k16 * L + j
                    for q in range(DF // L):
                        qsl = pl.ds(q * L, L)
                        gbufs[p][row, qsl] = gbufs[p][row, qsl] * v
                return 0

            lax.fori_loop(0, G // L, body, 0)

        fire_gather(0, 0)

        def group(g, _):
            ci = g * PP
            for pos in range(PP):
                cur = ci + pos
                nxt = cur + 1
                fp = pos % 2
                pn = (pos + 1) % 2

                @pl.when(jnp.logical_and(nxt >= 2, nxt < nch))
                def _():
                    swait(rowb, pn)

                @pl.when(nxt < nch)
                def _():
                    fire_gather(nxt, pn)

                gwait(fp)
                scale(cur, fp)
                fire_scatter(rowb, cur, fp)
            return 0

        lax.fori_loop(0, nch // PP, group, 0)
        swait(rowb, 0)
        swait(rowb, 1)

    run_spmm_bf(dcol, drow, dval, DGT, xb)
    plsc.subcore_barrier()

    for h in range(RPT // G):
        hsl = pl.ds(s * RPT + h * G, G)
        pltpu.sync_copy(acc.at[hsl], gb0)

        def rscale(k16, _, h=h):
            vv = rndbuf[pl.ds(h * G + k16 * L, L)]
            for j in range(L):
                v = vv[j]
                row = k16 * L + j
                for q in range(DF // L):
                    qsl = pl.ds(q * L, L)
                    gb0[row, qsl] = gb0[row, qsl] * v
            return 0

        lax.fori_loop(0, G // L, rscale, 0)
        pltpu.sync_copy(gb0, t_stage.at[pl.ds(c * N + s * RPT + h * G, G)])

    cN = c * N

    def bump(g, _):
        for j in range(G // L):
            jsl = pl.ds(j * L, L)
            dcol[g, jsl] = dcol[g, jsl] + cN
        return 0

    lax.fori_loop(0, DGT, bump, 0)
    pltpu.sync_copy(zeros.at[sl], acc.at[sl])
    plsc.subcore_barrier()

    run_spmm_f32(dcol, drow, dval, DGT, t_stage)
    plsc.subcore_barrier()

    @pl.when(c == 0)
    def _():
        pltpu.sync_copy(acc.at[sl], s2_out.at[sl])

    plsc.subcore_barrier()

    run_spmm_bf(acol, arow, aval, AGT, xb)
    plsc.subcore_barrier()
    pltpu.sync_copy(acc.at[sl], ap_out.at[c, sl])


_mega = pl.kernel(
    _mega_body,
    out_type=(
        jax.ShapeDtypeStruct((NC * N, DF), jnp.float32),
        jax.ShapeDtypeStruct((N, DF), jnp.float32),
        jax.ShapeDtypeStruct((NC, N, DF), jnp.float32),
    ),
    mesh=_mesh,
    compiler_params=pltpu.CompilerParams(use_tc_tiling_on_sc=False),
    scratch_types=[
        pltpu.VMEM_SHARED((N, DF), jnp.float32),
        pltpu.VMEM((DGT, G), jnp.int32),
        pltpu.VMEM((DGT, G), jnp.int32),
        pltpu.VMEM((PTD,), jnp.float32),
        pltpu.VMEM((AGT, G), jnp.int32),
        pltpu.VMEM((AGT, G), jnp.int32),
        pltpu.VMEM((PTA,), jnp.float32),
        pltpu.VMEM((RPT,), jnp.float32),
        pltpu.VMEM((G, DF), jnp.float32),
        pltpu.VMEM((G, DF), jnp.float32),
        pltpu.VMEM((G, DF // 2), jnp.int32),
        pltpu.VMEM((G, DF // 2), jnp.int32),
        pltpu.VMEM((G, DF // 2), jnp.int32),
        pltpu.VMEM((G, DF // 2), jnp.int32),
        pltpu.SemaphoreType.DMA,
        pltpu.SemaphoreType.DMA,
        pltpu.SemaphoreType.DMA,
        pltpu.SemaphoreType.DMA,
        pltpu.SemaphoreType.DMA,
        pltpu.SemaphoreType.DMA,
        pltpu.SemaphoreType.DMA,
    ],
)

_BLK = 512


def _combine_body(coef_ref, s2_ref, a_ref, h0_ref, w_ref, out_ref):
    th = coef_ref[0, 0]
    c1 = coef_ref[0, 1]
    c2 = coef_ref[0, 2]
    c3 = coef_ref[0, 3]
    sup = (c1 * s2_ref[...]
           + c2 * (a_ref[0] + a_ref[1])
           + c3 * h0_ref[...])
    out_ref[...] = th * jnp.dot(
        sup, w_ref[...], preferred_element_type=jnp.float32
    ) + (1.0 - th) * sup


def _combine(coefs, s2, ap, h0, w):
    return pl.pallas_call(
        _combine_body,
        grid=(N // _BLK,),
        in_specs=[
            pl.BlockSpec(memory_space=pltpu.MemorySpace.SMEM),
            pl.BlockSpec((_BLK, DF), lambda i: (i, 0)),
            pl.BlockSpec((NC, _BLK, DF), lambda i: (0, i, 0)),
            pl.BlockSpec((_BLK, DF), lambda i: (i, 0)),
            pl.BlockSpec((DF, DF), lambda i: (0, 0)),
        ],
        out_specs=pl.BlockSpec((_BLK, DF), lambda i: (i, 0)),
        out_shape=jax.ShapeDtypeStruct((N, DF), jnp.float32),
    )(coefs, s2, ap, h0, w)


def kernel(input, h0, adj_rows, adj_cols, adj_vals, d_rows, d_cols, d_vals,
           lamda, alpha, l, gamma, weight):
    x = input
    d1r = d_rows[1].reshape(-1, G)
    d1c = d_cols[1].reshape(-1, G)
    d1v = d_vals[1]
    rnd1 = jax.random.uniform(jax.random.key(42), (2 * N, 1), dtype=jnp.float32)[N:, 0]

    zeros = jnp.zeros((N, DF), jnp.float32)
    xh = x.astype(jnp.bfloat16).reshape(N, DF // 32, 2, 16)
    xb = jax.lax.bitcast_convert_type(
        jnp.moveaxis(xh, 2, 3), jnp.int32).reshape(N, DF // 2)
    _t, s2, ap = _mega(d1r, d1c, d1v,
                       adj_rows.reshape(-1, G), adj_cols.reshape(-1, G), adj_vals,
                       rnd1, xb, zeros)

    theta = jnp.log(lamda / l + 1.0)
    af = jnp.float32(alpha)
    gf = jnp.float32(gamma)
    c1 = (1.0 - af) * gf
    c2 = (1.0 - af) * (1.0 - gf)
    coefs = jnp.stack(
        [jnp.float32(theta), c1 - 2.0 * c2, c2, af]
    ).reshape(1, 4)

    return _combine(coefs, s2, ap, h0, weight)


# speedup vs baseline: 1.6957x; 1.6957x over previous
import jax
import jax.numpy as jnp
from jax import lax
from jax.experimental import pallas as pl
from jax.experimental.pallas import tpu as pltpu
from jax.experimental.pallas import tpu_sc as plsc

N = 4096
DF = 128
NNZ_ADJ = 131072
NNZ_D = 16384
NC = 2
NS = 16
NW = NC * NS
L = 16
G = 128
RPT = N // NS

PP = 4
PTD = NNZ_D // NS
PTA = NNZ_ADJ // NW
DGT = PTD // G
AGT = PTA // G
AQ0 = 24
AQ1 = 40

_mesh = plsc.VectorSubcoreMesh(
    core_axis_name="c", subcore_axis_name="s", num_cores=NC, num_subcores=NS
)


def _mega_body(d1r, d1c, d1v, a_r, a_c, a_v, rnd, x, zeros,
               t_stage, s2_out, ap_out,
               acc, dcol, drow, dval, acol, arow, aval, rndbuf,
               gb0, gb1, gb2, gb3,
               gsem0, gsem1, gsem2, gsem3,
               ssem0, ssem1, ssem2, ssem3, psem):
    c = lax.axis_index("c")
    s = lax.axis_index("s")
    wid = s * NC + c
    gbufs = (gb0, gb1, gb2, gb3)
    gsems = (gsem0, gsem1, gsem2, gsem3)
    ssems = (ssem0, ssem1, ssem2, ssem3)
    sl = pl.ds(s * RPT, RPT)
    abase = pl.multiple_of(
        jnp.where(c == 0, s * AQ0, NS * AQ0 + s * AQ1).astype(jnp.int32), 8)
    anch = jnp.where(c == 0, AQ0, AQ1)

    pre = [
        pltpu.async_copy(zeros.at[sl], acc.at[sl], psem),
        pltpu.async_copy(d1c.at[pl.ds(s * DGT, DGT)], dcol, psem),
        pltpu.async_copy(d1r.at[pl.ds(s * DGT, DGT)], drow, psem),
        pltpu.async_copy(d1v.at[pl.ds(pl.multiple_of(s * PTD, 8), PTD)], dval, psem),
        pltpu.async_copy(a_c.at[pl.ds(abase, AQ1)], acol, psem),
        pltpu.async_copy(a_r.at[pl.ds(abase, AQ1)], arow, psem),
        pltpu.async_copy(a_v.at[pl.ds(pl.multiple_of(abase * G, 8), AQ1 * G)], aval, psem),
        pltpu.async_copy(rnd.at[pl.ds(pl.multiple_of(s * RPT, 8), RPT)], rndbuf, psem),
    ]
    for d in pre:
        d.wait()
    plsc.subcore_barrier()

    def run_spmm(colb, rowb, valb, nch, table):
        if isinstance(nch, int):
            assert nch % PP == 0 and nch >= PP

        def fire_gather(idx, p):
            pltpu.async_copy(table.at[colb.at[idx]], gbufs[p], gsems[p])

        def gwait(p):
            pltpu.make_async_copy(table.at[colb.at[0]], gbufs[p], gsems[p]).wait()

        def fire_scatter(idx, p):
            pltpu.async_copy(gbufs[p], acc.at[rowb.at[idx]], ssems[p], add=True)

        def swait(p):
            pltpu.make_async_copy(gbufs[p], acc.at[rowb.at[0]], ssems[p]).wait()

        def scale(ci, p):
            def body(k16, _):
                vv = valb[pl.ds(ci * G + k16 * L, L)]
                for j in range(L):
                    v = vv[j]
                    row = k16 * L + j
                    for q in range(DF // L):
                        qsl = pl.ds(q * L, L)
                        gbufs[p][row, qsl] = gbufs[p][row, qsl] * v
                return 0

            lax.fori_loop(0, G // L, body, 0)

        fire_gather(0, 0)
        fire_gather(1, 1)

        def group(g, _):
            ci = g * PP
            for pos in range(PP):
                cur = ci + pos
                nxt = cur + 2
                pn = (pos + 2) % PP

                @pl.when(jnp.logical_and(nxt >= PP, nxt < nch))
                def _():
                    swait(pn)

                @pl.when(nxt < nch)
                def _():
                    fire_gather(nxt, pn)

                gwait(pos)
                scale(cur, pos)
                fire_scatter(cur, pos)
            return 0

        lax.fori_loop(0, nch // PP, group, 0)
        for p in range(PP):
            swait(p)

    @pl.when(c == 0)
    def _():
        run_spmm(dcol, drow, dval, DGT, x)
        plsc.subcore_barrier()

        for h in range(RPT // G):
            hsl = pl.ds(s * RPT + h * G, G)
            pltpu.sync_copy(acc.at[hsl], gb0)

            def rscale(k16, _, h=h):
                vv = rndbuf[pl.ds(h * G + k16 * L, L)]
                for j in range(L):
                    v = vv[j]
                    row = k16 * L + j
                    for q in range(DF // L):
                        qsl = pl.ds(q * L, L)
                        gb0[row, qsl] = gb0[row, qsl] * v
                return 0

            lax.fori_loop(0, G // L, rscale, 0)
            pltpu.sync_copy(gb0, t_stage.at[pl.ds(s * RPT + h * G, G)])

        pltpu.sync_copy(zeros.at[sl], acc.at[sl])
        plsc.subcore_barrier()

        run_spmm(dcol, drow, dval, DGT, t_stage)
        plsc.subcore_barrier()

        pltpu.sync_copy(acc.at[sl], s2_out.at[sl])
        plsc.subcore_barrier()

    run_spmm(acol, arow, aval, anch, x)
    plsc.subcore_barrier()
    pltpu.sync_copy(acc.at[sl], ap_out.at[c, sl])


_mega = pl.kernel(
    _mega_body,
    out_type=(
        jax.ShapeDtypeStruct((N, DF), jnp.float32),
        jax.ShapeDtypeStruct((N, DF), jnp.float32),
        jax.ShapeDtypeStruct((NC, N, DF), jnp.float32),
    ),
    mesh=_mesh,
    scratch_types=[
        pltpu.VMEM_SHARED((N, DF), jnp.float32),
        pltpu.VMEM((DGT, G), jnp.int32),
        pltpu.VMEM((DGT, G), jnp.int32),
        pltpu.VMEM((PTD,), jnp.float32),
        pltpu.VMEM((AQ1, G), jnp.int32),
        pltpu.VMEM((AQ1, G), jnp.int32),
        pltpu.VMEM((AQ1 * G,), jnp.float32),
        pltpu.VMEM((RPT,), jnp.float32),
        pltpu.VMEM((G, DF), jnp.float32),
        pltpu.VMEM((G, DF), jnp.float32),
        pltpu.VMEM((G, DF), jnp.float32),
        pltpu.VMEM((G, DF), jnp.float32),
        pltpu.SemaphoreType.DMA,
        pltpu.SemaphoreType.DMA,
        pltpu.SemaphoreType.DMA,
        pltpu.SemaphoreType.DMA,
        pltpu.SemaphoreType.DMA,
        pltpu.SemaphoreType.DMA,
        pltpu.SemaphoreType.DMA,
        pltpu.SemaphoreType.DMA,
        pltpu.SemaphoreType.DMA,
    ],
)

_BLK = 512


def _combine_body(coef_ref, s2_ref, a_ref, h0_ref, w_ref, out_ref):
    th = coef_ref[0, 0]
    c1 = coef_ref[0, 1]
    c2 = coef_ref[0, 2]
    c3 = coef_ref[0, 3]
    sup = (c1 * s2_ref[...]
           + c2 * (a_ref[0] + a_ref[1])
           + c3 * h0_ref[...])
    out_ref[...] = th * jnp.dot(
        sup, w_ref[...], preferred_element_type=jnp.float32
    ) + (1.0 - th) * sup


def _combine(coefs, s2, ap, h0, w):
    return pl.pallas_call(
        _combine_body,
        grid=(N // _BLK,),
        in_specs=[
            pl.BlockSpec(memory_space=pltpu.MemorySpace.SMEM),
            pl.BlockSpec((_BLK, DF), lambda i: (i, 0)),
            pl.BlockSpec((NC, _BLK, DF), lambda i: (0, i, 0)),
            pl.BlockSpec((_BLK, DF), lambda i: (i, 0)),
            pl.BlockSpec((DF, DF), lambda i: (0, 0)),
        ],
        out_specs=pl.BlockSpec((_BLK, DF), lambda i: (i, 0)),
        out_shape=jax.ShapeDtypeStruct((N, DF), jnp.float32),
    )(coefs, s2, ap, h0, w)


def kernel(input, h0, adj_rows, adj_cols, adj_vals, d_rows, d_cols, d_vals,
           lamda, alpha, l, gamma, weight):
    x = input
    d1r = d_rows[1].reshape(-1, G)
    d1c = d_cols[1].reshape(-1, G)
    d1v = d_vals[1]
    rnd1 = jax.random.uniform(jax.random.key(42), (2 * N, 1), dtype=jnp.float32)[N:, 0]

    zeros = jnp.zeros((N, DF), jnp.float32)
    _t, s2, ap = _mega(d1r, d1c, d1v,
                       adj_rows.reshape(-1, G), adj_cols.reshape(-1, G), adj_vals,
                       rnd1, x, zeros)

    theta = jnp.log(lamda / l + 1.0)
    af = jnp.float32(alpha)
    gf = jnp.float32(gamma)
    c1 = (1.0 - af) * gf
    c2 = (1.0 - af) * (1.0 - gf)
    coefs = jnp.stack(
        [jnp.float32(theta), c1 - c2, c2, af]
    ).reshape(1, 4)

    return _combine(coefs, s2, ap, h0, weight)


# speedup vs baseline: 1.7832x; 1.0516x over previous
import jax
import jax.numpy as jnp
from jax import lax
from jax.experimental import pallas as pl
from jax.experimental.pallas import tpu as pltpu
from jax.experimental.pallas import tpu_sc as plsc

N = 4096
DF = 128
NNZ_ADJ = 131072
NNZ_D = 16384
NC = 2
NS = 16
NW = NC * NS
L = 16
G = 128
RPT = N // NS

PP = 4
PTD = NNZ_D // NS
PTA = NNZ_ADJ // NW
DGT = PTD // G
AGT = PTA // G
AQ0 = 20
AQ1 = 44
ABUF = 56
TOTG = NNZ_ADJ // G

_mesh = plsc.VectorSubcoreMesh(
    core_axis_name="c", subcore_axis_name="s", num_cores=NC, num_subcores=NS
)


def _mega_body(d1r, d1c, d1v, a_r, a_c, a_v, rnd, x, zeros,
               t_stage, s2_out, ap_out,
               acc, dcol, drow, dval, acol, arow, aval, rndbuf,
               gb0, gb1, gb2, gb3,
               gsem0, gsem1, gsem2, gsem3,
               ssem0, ssem1, ssem2, ssem3, psem):
    c = lax.axis_index("c")
    s = lax.axis_index("s")
    wid = s * NC + c
    gbufs = (gb0, gb1, gb2, gb3)
    gsems = (gsem0, gsem1, gsem2, gsem3)
    ssems = (ssem0, ssem1, ssem2, ssem3)
    sl = pl.ds(s * RPT, RPT)
    abase = jnp.where(c == 0, s * AQ0, NS * AQ0 + s * AQ1).astype(jnp.int32)
    base8 = pl.multiple_of(
        jnp.minimum((abase // 8) * 8, TOTG - ABUF).astype(jnp.int32), 8)
    adel = abase - base8
    anch = jnp.where(c == 0, AQ0, AQ1)

    pre = [
        pltpu.async_copy(zeros.at[sl], acc.at[sl], psem),
        pltpu.async_copy(d1c.at[pl.ds(s * DGT, DGT)], dcol, psem),
        pltpu.async_copy(d1r.at[pl.ds(s * DGT, DGT)], drow, psem),
        pltpu.async_copy(d1v.at[pl.ds(pl.multiple_of(s * PTD, 8), PTD)], dval, psem),
        pltpu.async_copy(a_c.at[pl.ds(base8, ABUF)], acol, psem),
        pltpu.async_copy(a_r.at[pl.ds(base8, ABUF)], arow, psem),
        pltpu.async_copy(a_v.at[pl.ds(pl.multiple_of(base8 * G, 8), ABUF * G)], aval, psem),
        pltpu.async_copy(rnd.at[pl.ds(pl.multiple_of(s * RPT, 8), RPT)], rndbuf, psem),
    ]
    for d in pre:
        d.wait()
    plsc.subcore_barrier()

    def run_spmm(colb, rowb, valb, nch, table, goff=0):
        if isinstance(nch, int):
            assert nch % PP == 0 and nch >= PP

        def fire_gather(idx, p):
            pltpu.async_copy(table.at[colb.at[goff + idx]], gbufs[p], gsems[p])

        def gwait(p):
            pltpu.make_async_copy(table.at[colb.at[0]], gbufs[p], gsems[p]).wait()

        def fire_scatter(idx, p):
            pltpu.async_copy(gbufs[p], acc.at[rowb.at[goff + idx]], ssems[p], add=True)

        def swait(p):
            pltpu.make_async_copy(gbufs[p], acc.at[rowb.at[0]], ssems[p]).wait()

        def scale(ci, p):
            def body(k16, _):
                vv = valb[pl.ds(goff * G + ci * G + k16 * L, L)]
                for j in range(L):
                    v = vv[j]
                    row = k16 * L + j
                    for q in range(DF // L):
                        qsl = pl.ds(q * L, L)
                        gbufs[p][row, qsl] = gbufs[p][row, qsl] * v
                return 0

            lax.fori_loop(0, G // L, body, 0)

        fire_gather(0, 0)
        fire_gather(1, 1)

        def group(g, _):
            ci = g * PP
            for pos in range(PP):
                cur = ci + pos
                nxt = cur + 2
                pn = (pos + 2) % PP

                @pl.when(jnp.logical_and(nxt >= PP, nxt < nch))
                def _():
                    swait(pn)

                @pl.when(nxt < nch)
                def _():
                    fire_gather(nxt, pn)

                gwait(pos)
                scale(cur, pos)
                fire_scatter(cur, pos)
            return 0

        lax.fori_loop(0, nch // PP, group, 0)
        for p in range(PP):
            swait(p)

    @pl.when(c == 0)
    def _():
        run_spmm(dcol, drow, dval, DGT, x)
        plsc.subcore_barrier()

        for h in range(RPT // G):
            hsl = pl.ds(s * RPT + h * G, G)
            pltpu.sync_copy(acc.at[hsl], gb0)

            def rscale(k16, _, h=h):
                vv = rndbuf[pl.ds(h * G + k16 * L, L)]
                for j in range(L):
                    v = vv[j]
                    row = k16 * L + j
                    for q in range(DF // L):
                        qsl = pl.ds(q * L, L)
                        gb0[row, qsl] = gb0[row, qsl] * v
                return 0

            lax.fori_loop(0, G // L, rscale, 0)
            pltpu.sync_copy(gb0, t_stage.at[pl.ds(s * RPT + h * G, G)])

        pltpu.sync_copy(zeros.at[sl], acc.at[sl])
        plsc.subcore_barrier()

        run_spmm(dcol, drow, dval, DGT, t_stage)
        plsc.subcore_barrier()

        pltpu.sync_copy(acc.at[sl], s2_out.at[sl])
        plsc.subcore_barrier()

    run_spmm(acol, arow, aval, anch, x, goff=adel)
    plsc.subcore_barrier()
    pltpu.sync_copy(acc.at[sl], ap_out.at[c, sl])


_mega = pl.kernel(
    _mega_body,
    out_type=(
        jax.ShapeDtypeStruct((N, DF), jnp.float32),
        jax.ShapeDtypeStruct((N, DF), jnp.float32),
        jax.ShapeDtypeStruct((NC, N, DF), jnp.float32),
    ),
    mesh=_mesh,
    scratch_types=[
        pltpu.VMEM_SHARED((N, DF), jnp.float32),
        pltpu.VMEM((DGT, G), jnp.int32),
        pltpu.VMEM((DGT, G), jnp.int32),
        pltpu.VMEM((PTD,), jnp.float32),
        pltpu.VMEM((ABUF, G), jnp.int32),
        pltpu.VMEM((ABUF, G), jnp.int32),
        pltpu.VMEM((ABUF * G,), jnp.float32),
        pltpu.VMEM((RPT,), jnp.float32),
        pltpu.VMEM((G, DF), jnp.float32),
        pltpu.VMEM((G, DF), jnp.float32),
        pltpu.VMEM((G, DF), jnp.float32),
        pltpu.VMEM((G, DF), jnp.float32),
        pltpu.SemaphoreType.DMA,
        pltpu.SemaphoreType.DMA,
        pltpu.SemaphoreType.DMA,
        pltpu.SemaphoreType.DMA,
        pltpu.SemaphoreType.DMA,
        pltpu.SemaphoreType.DMA,
        pltpu.SemaphoreType.DMA,
        pltpu.SemaphoreType.DMA,
        pltpu.SemaphoreType.DMA,
    ],
)

_BLK = 512


def _combine_body(coef_ref, s2_ref, a_ref, h0_ref, w_ref, out_ref):
    th = coef_ref[0, 0]
    c1 = coef_ref[0, 1]
    c2 = coef_ref[0, 2]
    c3 = coef_ref[0, 3]
    sup = (c1 * s2_ref[...]
           + c2 * (a_ref[0] + a_ref[1])
           + c3 * h0_ref[...])
    out_ref[...] = th * jnp.dot(
        sup, w_ref[...], preferred_element_type=jnp.float32
    ) + (1.0 - th) * sup


def _combine(coefs, s2, ap, h0, w):
    return pl.pallas_call(
        _combine_body,
        grid=(N // _BLK,),
        in_specs=[
            pl.BlockSpec(memory_space=pltpu.MemorySpace.SMEM),
            pl.BlockSpec((_BLK, DF), lambda i: (i, 0)),
            pl.BlockSpec((NC, _BLK, DF), lambda i: (0, i, 0)),
            pl.BlockSpec((_BLK, DF), lambda i: (i, 0)),
            pl.BlockSpec((DF, DF), lambda i: (0, 0)),
        ],
        out_specs=pl.BlockSpec((_BLK, DF), lambda i: (i, 0)),
        out_shape=jax.ShapeDtypeStruct((N, DF), jnp.float32),
    )(coefs, s2, ap, h0, w)


def kernel(input, h0, adj_rows, adj_cols, adj_vals, d_rows, d_cols, d_vals,
           lamda, alpha, l, gamma, weight):
    x = input
    d1r = d_rows[1].reshape(-1, G)
    d1c = d_cols[1].reshape(-1, G)
    d1v = d_vals[1]
    rnd1 = jax.random.uniform(jax.random.key(42), (2 * N, 1), dtype=jnp.float32)[N:, 0]

    zeros = jnp.zeros((N, DF), jnp.float32)
    _t, s2, ap = _mega(d1r, d1c, d1v,
                       adj_rows.reshape(-1, G), adj_cols.reshape(-1, G), adj_vals,
                       rnd1, x, zeros)

    theta = jnp.log(lamda / l + 1.0)
    af = jnp.float32(alpha)
    gf = jnp.float32(gamma)
    c1 = (1.0 - af) * gf
    c2 = (1.0 - af) * (1.0 - gf)
    coefs = jnp.stack(
        [jnp.float32(theta), c1 - c2, c2, af]
    ).reshape(1, 4)

    return _combine(coefs, s2, ap, h0, weight)
